# baseline (speedup 1.0000x reference)
import functools

import jax
import jax.numpy as jnp
from jax import lax
from jax.experimental import pallas as pl
from jax.experimental.pallas import tpu as pltpu
from jax.experimental.pallas import tpu_sc as plsc

T = 4096
D = 768
E = 8
K = 2
BLK = 256
NB = T * K // BLK + E
PAD = NB * BLK
NTILES = 32
TPW = T // NTILES


def _lane_cumsum_excl(a, n):
    incl = a
    s = 1
    while s < n:
        incl = incl + jnp.concatenate(
            [jnp.zeros((a.shape[0], s), a.dtype), incl[:, :n - s]], axis=1)
        s *= 2
    return incl - a, incl[:, n - 1:n]


def _router_body(x_ref, wr_ref, br_ref, idx_ref, pb_ref, x16_ref):
    xb = x_ref[...]
    u = lax.bitcast_convert_type(
        xb.astype(jnp.bfloat16).astype(jnp.float32), jnp.int32)
    x16_ref[...] = (lax.shift_right_logical(u[:, :D // 2], 16)
                    | (u[:, D // 2:] & jnp.int32(-65536)))
    logits = jnp.dot(xb, wr_ref[...],
                     preferred_element_type=jnp.float32) + br_ref[...]
    m = jnp.max(logits, axis=1, keepdims=True)
    ex = jnp.exp(logits - m)
    probs = ex / jnp.sum(ex, axis=1, keepdims=True)

    iota = lax.broadcasted_iota(jnp.int32, (T, E), 1)
    m1 = jnp.max(probs, axis=1, keepdims=True)
    idx1 = jnp.min(jnp.where(probs == m1, iota, E), axis=1, keepdims=True)
    sel1 = iota == idx1
    probs_m = jnp.where(sel1, -1.0, probs)
    m2 = jnp.max(probs_m, axis=1, keepdims=True)
    idx2 = jnp.min(jnp.where(probs_m == m2, iota, E), axis=1, keepdims=True)
    sel2 = iota == idx2

    pb_ref[...] = jnp.concatenate(
        [jnp.broadcast_to(m1, (T, 16)), jnp.broadcast_to(m2, (T, 16))], axis=1)

    s0 = jnp.transpose(sel1.astype(jnp.float32))
    s1 = jnp.transpose(sel2.astype(jnp.float32))
    excl0, cnt0 = _lane_cumsum_excl(s0, T)
    excl1, cnt1 = _lane_cumsum_excl(s1, T)
    cnt = cnt0 + cnt1
    pc = jnp.floor((cnt + (BLK - 1)) / BLK) * BLK
    incl = pc
    for s in (1, 2, 4):
        incl = incl + jnp.concatenate(
            [jnp.zeros((s, 1), jnp.float32), incl[:E - s, :]], axis=0)
    pad_start = incl - pc

    dst0 = jnp.sum(s0 * (pad_start + excl0), axis=0, keepdims=True)
    dst1 = jnp.sum(s1 * (pad_start + cnt0 + excl1), axis=0, keepdims=True)

    psb = pad_start / BLK
    iob = lax.broadcasted_iota(jnp.int32, (E, 64), 1).astype(jnp.float32)
    bexp = jnp.sum((iob >= psb).astype(jnp.float32), axis=0, keepdims=True) - 1.0
    bexp_row = jnp.concatenate([bexp, jnp.zeros((1, T - 64), jnp.float32)], axis=1)

    idx_ref[...] = jnp.concatenate(
        [dst0, dst1, bexp_row, jnp.zeros((E - 3, T), jnp.float32)],
        axis=0).astype(jnp.int32)


def _router_call(x, Wr, br):
    return pl.pallas_call(
        _router_body,
        out_shape=(jax.ShapeDtypeStruct((E, T), jnp.int32),
                   jax.ShapeDtypeStruct((T, 32), jnp.float32),
                   jax.ShapeDtypeStruct((T, D // 2), jnp.int32)),
    )(x, Wr, br)


@functools.cache
def _dispatch_kernel():
    mesh = plsc.VectorSubcoreMesh(core_axis_name="c", subcore_axis_name="s")

    @functools.partial(
        pl.kernel, mesh=mesh,
        out_type=jax.ShapeDtypeStruct((PAD, D // 2), jnp.int32),
        scratch_types=[
            pltpu.VMEM((TPW, D // 2), jnp.int32),
            pltpu.VMEM((TPW,), jnp.int32),
            pltpu.VMEM((TPW,), jnp.int32),
            pltpu.SemaphoreType.DMA,
            pltpu.SemaphoreType.DMA,
        ],
    )
    def _dispatch(x_hbm, d0_hbm, d1_hbm, xs_hbm, xv, i0, i1, s0, s1):
        wid = lax.axis_index("s") * 2 + lax.axis_index("c")
        base = wid * TPW
        pltpu.sync_copy(x_hbm.at[pl.ds(base, TPW)], xv)
        pltpu.sync_copy(d0_hbm.at[pl.ds(base, TPW)], i0)
        pltpu.sync_copy(d1_hbm.at[pl.ds(base, TPW)], i1)
        c0 = pltpu.async_copy(xv, xs_hbm.at[i0], s0)
        c1 = pltpu.async_copy(xv, xs_hbm.at[i1], s1)
        c0.wait()
        c1.wait()

    return _dispatch


def _gmm_body(bexp_ref, xs_ref, we_ref, be_ref, ys_ref):
    e = bexp_ref[pl.program_id(0)]
    w = xs_ref[...]
    xlo = lax.bitcast_convert_type(lax.shift_left(w, 16), jnp.float32)
    xhi = lax.bitcast_convert_type(w & jnp.int32(-65536), jnp.float32)
    xb = jnp.concatenate([xlo, xhi], axis=1).astype(jnp.bfloat16)
    ys_ref[...] = jnp.dot(xb, we_ref[e],
                          preferred_element_type=jnp.float32) + be_ref[e]


def _gmm_call(bexp, xs, We16, be3):
    return pl.pallas_call(
        _gmm_body,
        grid_spec=pltpu.PrefetchScalarGridSpec(
            num_scalar_prefetch=1,
            grid=(NB,),
            in_specs=[
                pl.BlockSpec((BLK, D // 2), lambda b, br: (b, 0)),
                pl.BlockSpec((E, D, D), lambda b, br: (0, 0, 0)),
                pl.BlockSpec((E, 1, D), lambda b, br: (0, 0, 0)),
            ],
            out_specs=pl.BlockSpec((BLK, D), lambda b, br: (b, 0)),
        ),
        out_shape=jax.ShapeDtypeStruct((PAD, D), jnp.float32),
    )(bexp, xs, We16, be3)


CHUNK = 32


@functools.cache
def _combine_kernel():
    mesh = plsc.VectorSubcoreMesh(core_axis_name="c", subcore_axis_name="s")

    @functools.partial(
        pl.kernel, mesh=mesh,
        out_type=jax.ShapeDtypeStruct((T, D), jnp.float32),
        scratch_types=[
            pltpu.VMEM((2, CHUNK), jnp.int32),
            pltpu.VMEM((2, CHUNK), jnp.int32),
            pltpu.VMEM((2, CHUNK, 32), jnp.float32),
            pltpu.VMEM((2, CHUNK, D), jnp.float32),
            pltpu.VMEM((2, CHUNK, D), jnp.float32),
            pltpu.SemaphoreType.DMA,
            pltpu.SemaphoreType.DMA,
            pltpu.SemaphoreType.DMA,
            pltpu.SemaphoreType.DMA,
        ],
    )
    def _combine(ys_hbm, d0_hbm, d1_hbm, pb_hbm, out_hbm,
                 i0, i1, pbv, g0, g1, sa0, sa1, sb0, sb1):
        wid = lax.axis_index("s") * 2 + lax.axis_index("c")
        base = wid * TPW
        nch = TPW // CHUNK
        sems = ((sa0, sa1), (sb0, sb1))

        def fire(c):
            p = c % 2
            tb = base + c * CHUNK
            pltpu.sync_copy(d0_hbm.at[pl.ds(tb, CHUNK)], i0.at[p])
            pltpu.sync_copy(d1_hbm.at[pl.ds(tb, CHUNK)], i1.at[p])
            pltpu.sync_copy(pb_hbm.at[pl.ds(tb, CHUNK)], pbv.at[p])
            h0 = pltpu.async_copy(ys_hbm.at[i0.at[p]], g0.at[p], sems[p][0])
            h1 = pltpu.async_copy(ys_hbm.at[i1.at[p]], g1.at[p], sems[p][1])
            return h0, h1

        handles = [fire(0)]
        for c in range(nch):
            p = c % 2
            if c + 1 < nch:
                handles.append(fire(c + 1))
            h0, h1 = handles[c]
            h0.wait()
            h1.wait()

            def row(r, rcarry):
                b0 = pbv[p, r, pl.ds(0, 16)]
                b1 = pbv[p, r, pl.ds(16, 16)]
                for j in range(D // 16):
                    sl = pl.ds(j * 16, 16)
                    g0[p, r, sl] = b0 * g0[p, r, sl] + b1 * g1[p, r, sl]
                return rcarry

            lax.fori_loop(0, CHUNK, row, 0)
            pltpu.sync_copy(g0.at[p], out_hbm.at[pl.ds(base + c * CHUNK, CHUNK)])

    return _combine


@jax.jit
def kernel(x, Wr, br, We, be):
    idx_all, pb, x16p = _router_call(x, Wr, br)
    dst0 = idx_all[0]
    dst1 = idx_all[1]
    bexp = idx_all[2, :NB]
    We_b, _ = lax.optimization_barrier((We, dst0))
    We16 = We_b.astype(jnp.bfloat16)
    xs = _dispatch_kernel()(x16p, dst0, dst1)
    ys = _gmm_call(bexp, xs, We16, be.reshape(E, 1, D))
    return _combine_kernel()(ys, dst0, dst1, pb)


# speedup vs baseline: 1.0830x; 1.0830x over previous
import functools

import jax
import jax.numpy as jnp
from jax import lax
from jax.experimental import pallas as pl
from jax.experimental.pallas import tpu as pltpu
from jax.experimental.pallas import tpu_sc as plsc

T = 4096
D = 768
E = 8
K = 2
BLK = 256
NB = T * K // BLK + E
PAD = NB * BLK
NTILES = 32
TPW = T // NTILES


def _lane_cumsum_excl(a, n):
    incl = a
    s = 1
    while s < n:
        incl = incl + jnp.concatenate(
            [jnp.zeros((a.shape[0], s), a.dtype), incl[:, :n - s]], axis=1)
        s *= 2
    return incl - a, incl[:, n - 1:n]


def _router_body(x_ref, wr_ref, br_ref, idx_ref, pb_ref, x16_ref):
    xb = x_ref[...]
    u = lax.bitcast_convert_type(
        xb.astype(jnp.bfloat16).astype(jnp.float32), jnp.int32)
    x16_ref[...] = (lax.shift_right_logical(u[:, :D // 2], 16)
                    | (u[:, D // 2:] & jnp.int32(-65536)))
    logits = jnp.dot(xb, wr_ref[...],
                     preferred_element_type=jnp.float32) + br_ref[...]
    m = jnp.max(logits, axis=1, keepdims=True)
    ex = jnp.exp(logits - m)
    probs = ex / jnp.sum(ex, axis=1, keepdims=True)

    iota = lax.broadcasted_iota(jnp.int32, (T, E), 1)
    m1 = jnp.max(probs, axis=1, keepdims=True)
    idx1 = jnp.min(jnp.where(probs == m1, iota, E), axis=1, keepdims=True)
    sel1 = iota == idx1
    probs_m = jnp.where(sel1, -1.0, probs)
    m2 = jnp.max(probs_m, axis=1, keepdims=True)
    idx2 = jnp.min(jnp.where(probs_m == m2, iota, E), axis=1, keepdims=True)
    sel2 = iota == idx2

    pb_ref[...] = jnp.concatenate(
        [jnp.broadcast_to(m1, (T, 16)), jnp.broadcast_to(m2, (T, 16))], axis=1)

    s0 = jnp.transpose(sel1.astype(jnp.float32))
    s1 = jnp.transpose(sel2.astype(jnp.float32))
    excl0, cnt0 = _lane_cumsum_excl(s0, T)
    excl1, cnt1 = _lane_cumsum_excl(s1, T)
    cnt = cnt0 + cnt1
    pc = jnp.floor((cnt + (BLK - 1)) / BLK) * BLK
    incl = pc
    for s in (1, 2, 4):
        incl = incl + jnp.concatenate(
            [jnp.zeros((s, 1), jnp.float32), incl[:E - s, :]], axis=0)
    pad_start = incl - pc

    dst0 = jnp.sum(s0 * (pad_start + excl0), axis=0, keepdims=True)
    dst1 = jnp.sum(s1 * (pad_start + cnt0 + excl1), axis=0, keepdims=True)

    psb = pad_start / BLK
    iob = lax.broadcasted_iota(jnp.int32, (E, 64), 1).astype(jnp.float32)
    bexp = jnp.sum((iob >= psb).astype(jnp.float32), axis=0, keepdims=True) - 1.0
    bexp_row = jnp.concatenate([bexp, jnp.zeros((1, T - 64), jnp.float32)], axis=1)

    idx_ref[...] = jnp.concatenate(
        [dst0, dst1, bexp_row, jnp.zeros((E - 3, T), jnp.float32)],
        axis=0).astype(jnp.int32)


def _router_call(x, Wr, br):
    return pl.pallas_call(
        _router_body,
        out_shape=(jax.ShapeDtypeStruct((E, T), jnp.int32),
                   jax.ShapeDtypeStruct((T, 32), jnp.float32),
                   jax.ShapeDtypeStruct((T, D // 2), jnp.int32)),
    )(x, Wr, br)


@functools.cache
def _dispatch_kernel():
    mesh = plsc.VectorSubcoreMesh(core_axis_name="c", subcore_axis_name="s")

    @functools.partial(
        pl.kernel, mesh=mesh,
        out_type=jax.ShapeDtypeStruct((PAD, D // 2), jnp.int32),
        scratch_types=[
            pltpu.VMEM((TPW, D // 2), jnp.int32),
            pltpu.VMEM((TPW,), jnp.int32),
            pltpu.VMEM((TPW,), jnp.int32),
            pltpu.SemaphoreType.DMA,
            pltpu.SemaphoreType.DMA,
        ],
    )
    def _dispatch(x_hbm, d0_hbm, d1_hbm, xs_hbm, xv, i0, i1, s0, s1):
        wid = lax.axis_index("s") * 2 + lax.axis_index("c")
        base = wid * TPW
        pltpu.sync_copy(x_hbm.at[pl.ds(base, TPW)], xv)
        pltpu.sync_copy(d0_hbm.at[pl.ds(base, TPW)], i0)
        pltpu.sync_copy(d1_hbm.at[pl.ds(base, TPW)], i1)
        c0 = pltpu.async_copy(xv, xs_hbm.at[i0], s0)
        c1 = pltpu.async_copy(xv, xs_hbm.at[i1], s1)
        c0.wait()
        c1.wait()

    return _dispatch


def _gmm_body(bexp_ref, xs_ref, we_ref, be_ref, ys_ref):
    e = bexp_ref[pl.program_id(0)]
    w = xs_ref[...]
    xlo = lax.bitcast_convert_type(lax.shift_left(w, 16), jnp.float32)
    xhi = lax.bitcast_convert_type(w & jnp.int32(-65536), jnp.float32)
    xb = jnp.concatenate([xlo, xhi], axis=1).astype(jnp.bfloat16)
    ys_ref[...] = jnp.dot(xb, we_ref[e],
                          preferred_element_type=jnp.float32) + be_ref[e]


def _gmm_call(bexp, xs, We16, be3):
    return pl.pallas_call(
        _gmm_body,
        grid_spec=pltpu.PrefetchScalarGridSpec(
            num_scalar_prefetch=1,
            grid=(NB,),
            in_specs=[
                pl.BlockSpec((BLK, D // 2), lambda b, br: (b, 0)),
                pl.BlockSpec((E, D, D), lambda b, br: (0, 0, 0)),
                pl.BlockSpec((E, 1, D), lambda b, br: (0, 0, 0)),
            ],
            out_specs=pl.BlockSpec((BLK, D), lambda b, br: (b, 0)),
        ),
        out_shape=jax.ShapeDtypeStruct((PAD, D), jnp.float32),
    )(bexp, xs, We16, be3)


CHUNK = 64


@functools.cache
def _combine_kernel():
    mesh = plsc.VectorSubcoreMesh(core_axis_name="c", subcore_axis_name="s")

    @functools.partial(
        pl.kernel, mesh=mesh,
        out_type=jax.ShapeDtypeStruct((T, D), jnp.float32),
        scratch_types=[
            pltpu.VMEM((CHUNK,), jnp.int32),
            pltpu.VMEM((CHUNK,), jnp.int32),
            pltpu.VMEM((CHUNK, 32), jnp.float32),
            pltpu.VMEM((CHUNK, D), jnp.float32),
            pltpu.VMEM((CHUNK, D), jnp.float32),
            pltpu.SemaphoreType.DMA,
            pltpu.SemaphoreType.DMA,
        ],
    )
    def _combine(ys_hbm, d0_hbm, d1_hbm, pb_hbm, out_hbm,
                 i0, i1, pbv, g0, g1, s0, s1):
        wid = lax.axis_index("s") * 2 + lax.axis_index("c")
        base = wid * TPW

        def chunk(c, carry):
            tb = base + c * CHUNK
            pltpu.sync_copy(d0_hbm.at[pl.ds(tb, CHUNK)], i0)
            pltpu.sync_copy(d1_hbm.at[pl.ds(tb, CHUNK)], i1)
            pltpu.sync_copy(pb_hbm.at[pl.ds(tb, CHUNK)], pbv)
            c0 = pltpu.async_copy(ys_hbm.at[i0], g0, s0)
            c1 = pltpu.async_copy(ys_hbm.at[i1], g1, s1)
            c0.wait()
            c1.wait()

            def row(r, rcarry):
                b0 = pbv[r, pl.ds(0, 16)]
                b1 = pbv[r, pl.ds(16, 16)]
                for j in range(D // 16):
                    sl = pl.ds(j * 16, 16)
                    g0[r, sl] = b0 * g0[r, sl] + b1 * g1[r, sl]
                return rcarry

            lax.fori_loop(0, CHUNK, row, 0)
            pltpu.sync_copy(g0, out_hbm.at[pl.ds(tb, CHUNK)])
            return carry

        lax.fori_loop(0, TPW // CHUNK, chunk, 0)

    return _combine


@jax.jit
def kernel(x, Wr, br, We, be):
    idx_all, pb, x16p = _router_call(x, Wr, br)
    dst0 = idx_all[0]
    dst1 = idx_all[1]
    bexp = idx_all[2, :NB]
    We_b, _ = lax.optimization_barrier((We, dst0))
    We16 = We_b.astype(jnp.bfloat16)
    xs = _dispatch_kernel()(x16p, dst0, dst1)
    ys = _gmm_call(bexp, xs, We16, be.reshape(E, 1, D))
    return _combine_kernel()(ys, dst0, dst1, pb)
